# Initial kernel scaffold; baseline (speedup 1.0000x reference)
import jax
import jax.numpy as jnp
from jax.experimental import pallas as pl


def kernel(input_ids, item_codes, centroids):
    raise NotImplementedError("write your pallas kernel here")


# speedup vs baseline: 58.7823x; 58.7823x over previous
import functools
import jax
import jax.numpy as jnp
from jax import lax
from jax.experimental import pallas as pl
from jax.experimental.pallas import tpu as pltpu
from jax.experimental.pallas import tpu_sc as plsc

PQ_M = 8
SUB_EMB = 16
VALS_PER_DIM = 256
BATCH = 1024
SEQ_LEN = 200
EMB = PQ_M * SUB_EMB

NC, NS, L = 2, 16, 16
NW = NC * NS
TOTAL = BATCH * SEQ_LEN
PER_W = TOTAL // NW
CHUNK = 128
NCHUNK = PER_W // CHUNK
GROUPS = CHUNK * PQ_M // 128
T_PER_CHUNK = CHUNK * EMB // (128 * SUB_EMB)


def _body(ids_ref, codes_ref, cent_ref, out_ref,
          ids_v, codes_v, flat_v, rows_v, sem_codes, sem_rows):
    wid = lax.axis_index("s") * NC + lax.axis_index("c")

    iota = lax.iota(jnp.int32, L)
    row_pat = iota // PQ_M
    col_pat = lax.rem(iota, PQ_M)
    off_pat = col_pat * VALS_PER_DIM

    def chunk_body(k, carry):
        row = wid * NCHUNK + k
        pltpu.sync_copy(ids_ref.at[row], ids_v)
        pltpu.async_copy(codes_ref.at[ids_v], codes_v, sem_codes).wait()
        def idx_body(t, carry2):
            r0 = 2 * t
            rows16 = row_pat + r0
            codes16 = plsc.load_gather(codes_v, [rows16, col_pat])
            g = t // 8
            o = (t - g * 8) * L
            flat_v.at[g][pl.ds(o, L)] = codes16 + off_pat
            return carry2
        lax.fori_loop(0, CHUNK * PQ_M // L, idx_body, 0, unroll=8)
        copies = [
            pltpu.async_copy(cent_ref.at[flat_v.at[g]], rows_v.at[g], sem_rows)
            for g in range(GROUPS)
        ]
        for c in copies:
            c.wait()
        t0 = (wid * PER_W + k * CHUNK) * EMB // (128 * SUB_EMB)
        pltpu.sync_copy(rows_v, out_ref.at[pl.ds(t0, T_PER_CHUNK)])
        return carry

    lax.fori_loop(0, NCHUNK, chunk_body, 0)


@jax.jit
def _sc_call(ids2d, item_codes, cent2d):
    mesh = plsc.VectorSubcoreMesh(core_axis_name="c", subcore_axis_name="s")
    f = pl.kernel(
        _body,
        out_type=jax.ShapeDtypeStruct((TOTAL // SUB_EMB, 128, SUB_EMB),
                                      jnp.float32),
        mesh=mesh,
        scratch_types=[
            pltpu.VMEM((CHUNK,), jnp.int32),
            pltpu.VMEM((CHUNK, PQ_M), jnp.int32),
            pltpu.VMEM((GROUPS, 128), jnp.int32),
            pltpu.VMEM((GROUPS, 128, SUB_EMB), jnp.float32),
            pltpu.SemaphoreType.DMA,
            pltpu.SemaphoreType.DMA,
        ],
        compiler_params=pltpu.CompilerParams(use_tc_tiling_on_sc=False, needs_layout_passes=False),
    )
    return f(ids2d, item_codes, cent2d)


def kernel(input_ids, item_codes, centroids):
    ids2d = input_ids.reshape(TOTAL // 128, 128)
    cent2d = centroids.reshape(PQ_M * VALS_PER_DIM, SUB_EMB)
    out3d = _sc_call(ids2d, item_codes, cent2d)
    return out3d.reshape(BATCH, SEQ_LEN, EMB)


# speedup vs baseline: 66.5592x; 1.1323x over previous
import jax
import jax.numpy as jnp
from jax import lax
from jax.experimental import pallas as pl
from jax.experimental.pallas import tpu as pltpu
from jax.experimental.pallas import tpu_sc as plsc

PQ_M = 8
SUB_EMB = 16
VALS_PER_DIM = 256
BATCH = 1024
SEQ_LEN = 200
EMB = PQ_M * SUB_EMB

NC, NS, L = 2, 16, 16
NW = NC * NS
TOTAL = BATCH * SEQ_LEN
PER_W = TOTAL // NW
CHUNK = 128
NCHUNK = PER_W // CHUNK
GROUPS = CHUNK * PQ_M // 128
T_PER_CHUNK = CHUNK // SUB_EMB


def _body(ids_ref, codes_ref, cent_ref, out_ref,
          ids_v, codes_v, flat_v, rows_v,
          sem_codes, sem_rows0, sem_rows1, sem_out0, sem_out1):
    wid = lax.axis_index("s") * NC + lax.axis_index("c")
    sem_rows = (sem_rows0, sem_rows1)
    sem_out = (sem_out0, sem_out1)

    iota = lax.iota(jnp.int32, L)
    row_pat = iota // PQ_M
    col_pat = lax.rem(iota, PQ_M)
    off_pat = col_pat * VALS_PER_DIM

    def t_base(k):
        return wid * (PER_W // SUB_EMB) + k * T_PER_CHUNK

    def front(k, p):
        pltpu.sync_copy(ids_ref.at[wid * NCHUNK + k], ids_v.at[p])
        pltpu.async_copy(codes_ref.at[ids_v.at[p]], codes_v.at[p],
                         sem_codes).wait()

        def idx_body(t, c):
            rows16 = row_pat + 2 * t
            codes16 = plsc.load_gather(codes_v.at[p], [rows16, col_pat])
            g = t // 8
            o = (t - g * 8) * L
            flat_v.at[p].at[g][pl.ds(o, L)] = codes16 + off_pat
            return c

        lax.fori_loop(0, CHUNK * PQ_M // L, idx_body, 0, unroll=8)

    def fire_gathers(p):
        for g in range(GROUPS):
            pltpu.async_copy(cent_ref.at[flat_v.at[p].at[g]],
                             rows_v.at[p].at[g], sem_rows[p])

    def drain_gathers(p):
        pltpu.make_async_copy(out_ref.at[pl.ds(0, T_PER_CHUNK)],
                              rows_v.at[p], sem_rows[p]).wait()

    def fire_out(k, p):
        pltpu.async_copy(rows_v.at[p],
                         out_ref.at[pl.ds(t_base(k), T_PER_CHUNK)],
                         sem_out[p])

    def drain_out(p):
        pltpu.make_async_copy(rows_v.at[p],
                              out_ref.at[pl.ds(0, T_PER_CHUNK)],
                              sem_out[p]).wait()

    def pair_body(kk, carry):
        for p in (0, 1):
            k = 2 * kk + p

            @pl.when(k >= 2)
            def _():
                drain_out(p)

            front(k, p)

            @pl.when(k >= 1)
            def _():
                drain_gathers(1 - p)
                fire_out(k - 1, 1 - p)

            fire_gathers(p)
        return carry

    lax.fori_loop(0, NCHUNK // 2, pair_body, 0)
    drain_gathers(1)
    fire_out(NCHUNK - 1, 1)
    drain_out(0)
    drain_out(1)


@jax.jit
def _sc_call(ids2d, item_codes, cent2d):
    mesh = plsc.VectorSubcoreMesh(core_axis_name="c", subcore_axis_name="s")
    f = pl.kernel(
        _body,
        out_type=jax.ShapeDtypeStruct((TOTAL // SUB_EMB, 128, SUB_EMB),
                                      jnp.float32),
        mesh=mesh,
        scratch_types=[
            pltpu.VMEM((2, CHUNK), jnp.int32),
            pltpu.VMEM((2, CHUNK, PQ_M), jnp.int32),
            pltpu.VMEM((2, GROUPS, 128), jnp.int32),
            pltpu.VMEM((2, GROUPS, 128, SUB_EMB), jnp.float32),
            pltpu.SemaphoreType.DMA,
            pltpu.SemaphoreType.DMA,
            pltpu.SemaphoreType.DMA,
            pltpu.SemaphoreType.DMA,
            pltpu.SemaphoreType.DMA,
        ],
        compiler_params=pltpu.CompilerParams(use_tc_tiling_on_sc=False,
                                             needs_layout_passes=False),
    )
    return f(ids2d, item_codes, cent2d)


def kernel(input_ids, item_codes, centroids):
    ids2d = input_ids.reshape(TOTAL // 128, 128)
    cent2d = centroids.reshape(PQ_M * VALS_PER_DIM, SUB_EMB)
    out3d = _sc_call(ids2d, item_codes, cent2d)
    return out3d.reshape(BATCH, SEQ_LEN, EMB)
